# baseline (speedup 1.0000x reference)
import functools

import jax
import jax.numpy as jnp
from jax import lax
from jax.experimental import pallas as pl
from jax.experimental.pallas import tpu as pltpu
from jax.experimental.pallas import tpu_sc as plsc

N = 10000
E = 320000
IN = 128
H = 256
OUT = 128
L = 3
G = 128

QTR = H // 4
NQ = 4
NS = 16
NC = 2
CHUNK = 128
RING = 2
SUP = 16
PAIR = 2 * SUP
EPAD = ((E + NS * CHUNK * PAIR - 1) // (NS * CHUNK * PAIR)) * (NS * CHUNK * PAIR)
EDGES_PER_TILE = EPAD // NS
CHUNKS_PER_TILE = EDGES_PER_TILE // CHUNK
NPAIR = CHUNKS_PER_TILE // PAIR
NROWS_IDX = EPAD // CHUNK
ACC_ROWS = 10048
ROWS_PER_TILE = 640

BM = 400
NBLK = N // BM


def _sc_agg_body(cur_q, src_all, dst_all, out, srcv, acc, sidx, didx, rows,
                 gsem, ssem):
    c = lax.axis_index("c")
    s = lax.axis_index("s")
    row0 = s * ROWS_PER_TILE
    last = N - (NS - 1) * ROWS_PER_TILE
    crow0 = s * CHUNKS_PER_TILE

    PROBE_HBM_GATHER = False

    for pss in range(2):
        q = 2 * c + pss
        off = q * N
        gtab = cur_q if PROBE_HBM_GATHER else srcv

        def stage(ib, sg):
            r0 = crow0 + sg * SUP
            pltpu.sync_copy(src_all.at[pl.ds(r0, SUP)], sidx.at[ib])
            pltpu.sync_copy(dst_all.at[pl.ds(r0, SUP)], didx.at[ib])
            if PROBE_HBM_GATHER:
                for t in range(SUP):
                    for k in range(CHUNK // 16):
                        sl = pl.ds(k * 16, 16)
                        sidx[ib, t, sl] = sidx[ib, t, sl] + off

        def fire_g(ib, t, r):
            pltpu.async_copy(gtab.at[sidx.at[ib, t]], rows.at[r], gsem.at[r])

        def wait_g(ib, t, r):
            pltpu.make_async_copy(gtab.at[sidx.at[ib, t]], rows.at[r],
                                  gsem.at[r]).wait()

        def fire_s(ib, t, r):
            pltpu.async_copy(rows.at[r], acc.at[didx.at[ib, t]], ssem.at[r],
                             add=True)

        def wait_s(ib, t, r):
            pltpu.make_async_copy(rows.at[r], acc.at[didx.at[ib, t]],
                                  ssem.at[r]).wait()

        with jax.named_scope("sc_init"):
            @pl.when(s < NS - 1)
            def _():
                sl_in = pl.ds(q * N + row0, ROWS_PER_TILE)
                pltpu.sync_copy(cur_q.at[sl_in],
                                srcv.at[pl.ds(row0, ROWS_PER_TILE)])
                pltpu.sync_copy(cur_q.at[sl_in],
                                acc.at[pl.ds(row0, ROWS_PER_TILE)])

            @pl.when(s == NS - 1)
            def _():
                sl_in = pl.ds(q * N + (NS - 1) * ROWS_PER_TILE, last)
                pltpu.sync_copy(cur_q.at[sl_in],
                                srcv.at[pl.ds((NS - 1) * ROWS_PER_TILE, last)])
                pltpu.sync_copy(cur_q.at[sl_in],
                                acc.at[pl.ds((NS - 1) * ROWS_PER_TILE, last)])

            plsc.subcore_barrier()

        with jax.named_scope("sc_stage0"):
            stage(0, 0)
        for j in range(RING - 1):
            fire_g(0, j, j % RING)

        def pair(p, carry):
            for ib in range(2):
                for t in range(SUP):
                    k = ib * SUP + t
                    r = k % RING
                    wait_g(ib, t, r)
                    fire_s(ib, t, r)
                    r2 = (k + RING - 1) % RING
                    if k == 0:
                        @pl.when(p > 0)
                        def _():
                            wait_s(0, 0, r2)
                    else:
                        wait_s(ib, t, r2)
                    kn = k + RING - 1
                    if kn < PAIR:
                        fire_g(kn // SUP, kn % SUP, r2)
                    else:
                        kw = kn - PAIR

                        @pl.when(p < NPAIR - 1)
                        def _():
                            fire_g(kw // SUP, kw % SUP, r2)
                    if t == RING - 1:
                        if ib == 0:
                            stage(1, 2 * p + 1)
                        else:
                            @pl.when(p < NPAIR - 1)
                            def _():
                                stage(0, 2 * p + 2)
            return carry

        with jax.named_scope("sc_edge_loop"):
            lax.fori_loop(0, NPAIR, pair, 0)
            wait_s(1, SUP - 1, (PAIR - 1) % RING)

        with jax.named_scope("sc_copyout"):
            plsc.subcore_barrier()

            @pl.when(s < NS - 1)
            def _():
                pltpu.sync_copy(acc.at[pl.ds(row0, ROWS_PER_TILE)],
                                out.at[pl.ds(q * N + row0, ROWS_PER_TILE)])

            @pl.when(s == NS - 1)
            def _():
                pltpu.sync_copy(
                    acc.at[pl.ds((NS - 1) * ROWS_PER_TILE, last)],
                    out.at[pl.ds(q * N + (NS - 1) * ROWS_PER_TILE, last)])


@functools.cache
def _sc_agg_kernel():
    return pl.kernel(
        _sc_agg_body,
        out_type=jax.ShapeDtypeStruct((NQ * N, QTR), jnp.float32),
        mesh=plsc.VectorSubcoreMesh(core_axis_name="c", subcore_axis_name="s",
                                    num_cores=NC, num_subcores=NS),
        scratch_types=[
            pltpu.VMEM_SHARED((N, QTR), jnp.float32),
            pltpu.VMEM_SHARED((ACC_ROWS, QTR), jnp.float32),
            pltpu.VMEM((2, SUP, CHUNK), jnp.int32),
            pltpu.VMEM((2, SUP, CHUNK), jnp.int32),
            pltpu.VMEM((RING, CHUNK, QTR), jnp.float32),
            pltpu.SemaphoreType.DMA((RING,)),
            pltpu.SemaphoreType.DMA((RING,)),
        ],
        compiler_params=pltpu.CompilerParams(use_tc_tiling_on_sc=False),
    )


def _sc_agg(cur_q, src_all, dst_all):
    return _sc_agg_kernel()(cur_q, src_all, dst_all)


def _emb_body(x_ref, w_ref, b_ref, o_ref):
    o_ref[...] = jnp.dot(x_ref[...], w_ref[...],
                         preferred_element_type=jnp.float32) + b_ref[...]


def _mlp_body(m_ref, w1_ref, b1_ref, w2_ref, b2_ref, m2_ref, st_ref):
    m = m_ref[...]
    h = jnp.maximum(jnp.dot(m, w1_ref[...],
                            preferred_element_type=jnp.float32) + b1_ref[...], 0.0)
    m2 = jnp.dot(h, w2_ref[...], preferred_element_type=jnp.float32) + b2_ref[...]
    m2_ref[...] = m2
    st = jnp.concatenate([jnp.sum(m2, 0, keepdims=True),
                          jnp.sum(m2 * m2, 0, keepdims=True)], axis=0)
    i = pl.program_id(0)

    @pl.when(i == 0)
    def _():
        st_ref[...] = st

    @pl.when(i > 0)
    def _():
        st_ref[...] = st_ref[...] + st


def _bn_pool_body(first, m2_ref, cur_ref, st_ref, g_ref, bb_ref, lw_ref,
                  pin_ref, batch_ref, cur_out_ref, pool_ref):
    i = pl.program_id(0)
    st = st_ref[...]
    mean = st[0:1, :] * (1.0 / N)
    var = st[1:2, :] * (1.0 / N) - mean * mean
    inv = lax.rsqrt(var + 1e-5)
    xa = jnp.maximum((m2_ref[...] - mean) * (inv * g_ref[...]) + bb_ref[...], 0.0)
    curn = xa + cur_ref[...]
    cur_out_ref[...] = curn
    z = jnp.dot(curn, lw_ref[...], preferred_element_type=jnp.float32)
    bt = batch_ref[0, 0, :]
    oh = (bt[:, None] == lax.broadcasted_iota(jnp.int32, (BM, G), 1)
          ).astype(jnp.float32)
    contrib = lax.dot_general(oh, z, (((0,), (0,)), ((), ())),
                              preferred_element_type=jnp.float32)

    @pl.when(i == 0)
    def _():
        if first:
            base = jnp.broadcast_to(jnp.sum(pin_ref[...], 0, keepdims=True),
                                    (G, OUT))
        else:
            base = pin_ref[...]
        pool_ref[...] = base + contrib

    @pl.when(i > 0)
    def _():
        pool_ref[...] = pool_ref[...] + contrib


def _emb_call(x, w, b):
    return pl.pallas_call(
        _emb_body,
        grid=(NBLK,),
        in_specs=[
            pl.BlockSpec((BM, IN), lambda i: (i, 0)),
            pl.BlockSpec((IN, H), lambda i: (0, 0)),
            pl.BlockSpec((1, H), lambda i: (0, 0)),
        ],
        out_specs=pl.BlockSpec((BM, H), lambda i: (i, 0)),
        out_shape=jax.ShapeDtypeStruct((N, H), jnp.float32),
    )(x, w, b)


def _mlp_call(m, w1, b1, w2, b2):
    return pl.pallas_call(
        _mlp_body,
        grid=(NBLK,),
        in_specs=[
            pl.BlockSpec((BM, H), lambda i: (i, 0)),
            pl.BlockSpec((H, H), lambda i: (0, 0)),
            pl.BlockSpec((1, H), lambda i: (0, 0)),
            pl.BlockSpec((H, H), lambda i: (0, 0)),
            pl.BlockSpec((1, H), lambda i: (0, 0)),
        ],
        out_specs=[
            pl.BlockSpec((BM, H), lambda i: (i, 0)),
            pl.BlockSpec((2, H), lambda i: (0, 0)),
        ],
        out_shape=[
            jax.ShapeDtypeStruct((N, H), jnp.float32),
            jax.ShapeDtypeStruct((2, H), jnp.float32),
        ],
    )(m, w1, b1, w2, b2)


def _bn_pool_call(first, m2, cur, st, g, bb, lw, pin, batch_r):
    pin_spec = (pl.BlockSpec((L, OUT), lambda i: (0, 0)) if first
                else pl.BlockSpec((G, OUT), lambda i: (0, 0)))
    return pl.pallas_call(
        functools.partial(_bn_pool_body, first),
        grid=(NBLK,),
        in_specs=[
            pl.BlockSpec((BM, H), lambda i: (i, 0)),
            pl.BlockSpec((BM, H), lambda i: (i, 0)),
            pl.BlockSpec((2, H), lambda i: (0, 0)),
            pl.BlockSpec((1, H), lambda i: (0, 0)),
            pl.BlockSpec((1, H), lambda i: (0, 0)),
            pl.BlockSpec((H, OUT), lambda i: (0, 0)),
            pin_spec,
            pl.BlockSpec((1, 1, BM), lambda i: (i, 0, 0)),
        ],
        out_specs=[
            pl.BlockSpec((BM, H), lambda i: (i, 0)),
            pl.BlockSpec((G, OUT), lambda i: (0, 0)),
        ],
        out_shape=[
            jax.ShapeDtypeStruct((N, H), jnp.float32),
            jax.ShapeDtypeStruct((G, OUT), jnp.float32),
        ],
    )(m2, cur, st, g, bb, lw, pin, batch_r)


def _quarter_layout(cur):
    return cur.reshape(N, NQ, QTR).transpose(1, 0, 2).reshape(NQ * N, QTR)


def _unquarter_layout(m_q):
    return m_q.reshape(NQ, N, QTR).transpose(1, 0, 2).reshape(N, H)


def kernel(x, edge_index, cycle_index, batch, W_emb, b_emb, conv_W1, conv_b1,
           conv_W2, conv_b2, bn_g, bn_b, lin_W, lin_b):
    src = edge_index[0].astype(jnp.int32)
    dst = edge_index[1].astype(jnp.int32)
    pad = EPAD - E
    src_p = jnp.concatenate([src, jnp.zeros((pad,), jnp.int32)])
    dst_p = jnp.concatenate([dst, jnp.full((pad,), N, jnp.int32)])
    src_all = src_p.reshape(NROWS_IDX, CHUNK)
    dst_all = dst_p.reshape(NROWS_IDX, CHUNK)
    batch_r = batch.astype(jnp.int32).reshape(NBLK, 1, BM)

    cur = _emb_call(x, W_emb, b_emb.reshape(1, H))

    pool = lin_b
    for i in range(L):
        cur_q = _quarter_layout(cur)
        m = _unquarter_layout(_sc_agg(cur_q, src_all, dst_all))
        m2, st = _mlp_call(m, conv_W1[i], conv_b1[i].reshape(1, H),
                           conv_W2[i], conv_b2[i].reshape(1, H))
        cur, pool = _bn_pool_call(i == 0, m2, cur, st,
                                  bn_g[i].reshape(1, H), bn_b[i].reshape(1, H),
                                  lin_W[i], pool, batch_r)
    return pool


# speedup vs baseline: 1.1112x; 1.1112x over previous
import functools

import jax
import jax.numpy as jnp
from jax import lax
from jax.experimental import pallas as pl
from jax.experimental.pallas import tpu as pltpu
from jax.experimental.pallas import tpu_sc as plsc

N = 10000
E = 320000
IN = 128
H = 256
OUT = 128
L = 3
G = 128

QTR = H // 4
NQ = 4
NS = 16
NC = 2
CHUNK = 128
RING = 4
SUP = 16
PAIR = 2 * SUP
EPAD = ((E + NS * CHUNK * PAIR - 1) // (NS * CHUNK * PAIR)) * (NS * CHUNK * PAIR)
EDGES_PER_TILE = EPAD // NS
CHUNKS_PER_TILE = EDGES_PER_TILE // CHUNK
NPAIR = CHUNKS_PER_TILE // PAIR
NROWS_IDX = EPAD // CHUNK
ACC_ROWS = 10048
ROWS_PER_TILE = 640

BM = 400
NBLK = N // BM


def _sc_agg_body(cur_q, src_all, dst_all, out, srcv, acc, sidx, didx, rows,
                 gsem, ssem):
    c = lax.axis_index("c")
    s = lax.axis_index("s")
    row0 = s * ROWS_PER_TILE
    last = N - (NS - 1) * ROWS_PER_TILE
    crow0 = s * CHUNKS_PER_TILE

    PROBE_HBM_GATHER = False

    for pss in range(2):
        q = 2 * c + pss
        off = q * N
        gtab = cur_q if PROBE_HBM_GATHER else srcv

        def stage(ib, sg):
            r0 = crow0 + sg * SUP
            pltpu.sync_copy(src_all.at[pl.ds(r0, SUP)], sidx.at[ib])
            pltpu.sync_copy(dst_all.at[pl.ds(r0, SUP)], didx.at[ib])
            if PROBE_HBM_GATHER:
                for t in range(SUP):
                    for k in range(CHUNK // 16):
                        sl = pl.ds(k * 16, 16)
                        sidx[ib, t, sl] = sidx[ib, t, sl] + off

        def fire_g(ib, t, r):
            pltpu.async_copy(gtab.at[sidx.at[ib, t]], rows.at[r], gsem.at[r])

        def wait_g(ib, t, r):
            pltpu.make_async_copy(gtab.at[sidx.at[ib, t]], rows.at[r],
                                  gsem.at[r]).wait()

        def fire_s(ib, t, r):
            pltpu.async_copy(rows.at[r], acc.at[didx.at[ib, t]], ssem.at[r],
                             add=True)

        def wait_s(ib, t, r):
            pltpu.make_async_copy(rows.at[r], acc.at[didx.at[ib, t]],
                                  ssem.at[r]).wait()

        with jax.named_scope("sc_init"):
            @pl.when(s < NS - 1)
            def _():
                sl_in = pl.ds(q * N + row0, ROWS_PER_TILE)
                pltpu.sync_copy(cur_q.at[sl_in],
                                srcv.at[pl.ds(row0, ROWS_PER_TILE)])
                pltpu.sync_copy(cur_q.at[sl_in],
                                acc.at[pl.ds(row0, ROWS_PER_TILE)])

            @pl.when(s == NS - 1)
            def _():
                sl_in = pl.ds(q * N + (NS - 1) * ROWS_PER_TILE, last)
                pltpu.sync_copy(cur_q.at[sl_in],
                                srcv.at[pl.ds((NS - 1) * ROWS_PER_TILE, last)])
                pltpu.sync_copy(cur_q.at[sl_in],
                                acc.at[pl.ds((NS - 1) * ROWS_PER_TILE, last)])

            plsc.subcore_barrier()

        with jax.named_scope("sc_stage0"):
            stage(0, 0)
        for j in range(RING - 1):
            fire_g(0, j, j % RING)

        def pair(p, carry):
            for ib in range(2):
                for t in range(SUP):
                    k = ib * SUP + t
                    r = k % RING
                    wait_g(ib, t, r)
                    fire_s(ib, t, r)
                    r2 = (k + RING - 1) % RING
                    if k == 0:
                        @pl.when(p > 0)
                        def _():
                            wait_s(0, 0, r2)
                    else:
                        wait_s(ib, t, r2)
                    kn = k + RING - 1
                    if kn < PAIR:
                        fire_g(kn // SUP, kn % SUP, r2)
                    else:
                        kw = kn - PAIR

                        @pl.when(p < NPAIR - 1)
                        def _():
                            fire_g(kw // SUP, kw % SUP, r2)
                    if t == RING - 1:
                        if ib == 0:
                            stage(1, 2 * p + 1)
                        else:
                            @pl.when(p < NPAIR - 1)
                            def _():
                                stage(0, 2 * p + 2)
            return carry

        with jax.named_scope("sc_edge_loop"):
            lax.fori_loop(0, NPAIR, pair, 0)
            wait_s(1, SUP - 1, (PAIR - 1) % RING)

        with jax.named_scope("sc_copyout"):
            plsc.subcore_barrier()

            @pl.when(s < NS - 1)
            def _():
                pltpu.sync_copy(acc.at[pl.ds(row0, ROWS_PER_TILE)],
                                out.at[pl.ds(q * N + row0, ROWS_PER_TILE)])

            @pl.when(s == NS - 1)
            def _():
                pltpu.sync_copy(
                    acc.at[pl.ds((NS - 1) * ROWS_PER_TILE, last)],
                    out.at[pl.ds(q * N + (NS - 1) * ROWS_PER_TILE, last)])


@functools.cache
def _sc_agg_kernel():
    return pl.kernel(
        _sc_agg_body,
        out_type=jax.ShapeDtypeStruct((NQ * N, QTR), jnp.float32),
        mesh=plsc.VectorSubcoreMesh(core_axis_name="c", subcore_axis_name="s",
                                    num_cores=NC, num_subcores=NS),
        scratch_types=[
            pltpu.VMEM_SHARED((N, QTR), jnp.float32),
            pltpu.VMEM_SHARED((ACC_ROWS, QTR), jnp.float32),
            pltpu.VMEM((2, SUP, CHUNK), jnp.int32),
            pltpu.VMEM((2, SUP, CHUNK), jnp.int32),
            pltpu.VMEM((RING, CHUNK, QTR), jnp.float32),
            pltpu.SemaphoreType.DMA((RING,)),
            pltpu.SemaphoreType.DMA((RING,)),
        ],
        compiler_params=pltpu.CompilerParams(use_tc_tiling_on_sc=False),
    )


def _sc_agg(cur_q, src_all, dst_all):
    return _sc_agg_kernel()(cur_q, src_all, dst_all)


def _emb_body(x_ref, w_ref, b_ref, o_ref):
    o_ref[...] = jnp.dot(x_ref[...], w_ref[...],
                         preferred_element_type=jnp.float32) + b_ref[...]


def _mlp_body(m_ref, w1_ref, b1_ref, w2_ref, b2_ref, m2_ref, st_ref):
    m = m_ref[...]
    h = jnp.maximum(jnp.dot(m, w1_ref[...],
                            preferred_element_type=jnp.float32) + b1_ref[...], 0.0)
    m2 = jnp.dot(h, w2_ref[...], preferred_element_type=jnp.float32) + b2_ref[...]
    m2_ref[...] = m2
    st = jnp.concatenate([jnp.sum(m2, 0, keepdims=True),
                          jnp.sum(m2 * m2, 0, keepdims=True)], axis=0)
    i = pl.program_id(0)

    @pl.when(i == 0)
    def _():
        st_ref[...] = st

    @pl.when(i > 0)
    def _():
        st_ref[...] = st_ref[...] + st


def _bn_pool_body(first, m2_ref, cur_ref, st_ref, g_ref, bb_ref, lw_ref,
                  pin_ref, batch_ref, cur_out_ref, pool_ref):
    i = pl.program_id(0)
    st = st_ref[...]
    mean = st[0:1, :] * (1.0 / N)
    var = st[1:2, :] * (1.0 / N) - mean * mean
    inv = lax.rsqrt(var + 1e-5)
    xa = jnp.maximum((m2_ref[...] - mean) * (inv * g_ref[...]) + bb_ref[...], 0.0)
    curn = xa + cur_ref[...]
    cur_out_ref[...] = curn
    z = jnp.dot(curn, lw_ref[...], preferred_element_type=jnp.float32)
    bt = batch_ref[0, 0, :]
    oh = (bt[:, None] == lax.broadcasted_iota(jnp.int32, (BM, G), 1)
          ).astype(jnp.float32)
    contrib = lax.dot_general(oh, z, (((0,), (0,)), ((), ())),
                              preferred_element_type=jnp.float32)

    @pl.when(i == 0)
    def _():
        if first:
            base = jnp.broadcast_to(jnp.sum(pin_ref[...], 0, keepdims=True),
                                    (G, OUT))
        else:
            base = pin_ref[...]
        pool_ref[...] = base + contrib

    @pl.when(i > 0)
    def _():
        pool_ref[...] = pool_ref[...] + contrib


def _emb_call(x, w, b):
    return pl.pallas_call(
        _emb_body,
        grid=(NBLK,),
        in_specs=[
            pl.BlockSpec((BM, IN), lambda i: (i, 0)),
            pl.BlockSpec((IN, H), lambda i: (0, 0)),
            pl.BlockSpec((1, H), lambda i: (0, 0)),
        ],
        out_specs=pl.BlockSpec((BM, H), lambda i: (i, 0)),
        out_shape=jax.ShapeDtypeStruct((N, H), jnp.float32),
    )(x, w, b)


def _mlp_call(m, w1, b1, w2, b2):
    return pl.pallas_call(
        _mlp_body,
        grid=(NBLK,),
        in_specs=[
            pl.BlockSpec((BM, H), lambda i: (i, 0)),
            pl.BlockSpec((H, H), lambda i: (0, 0)),
            pl.BlockSpec((1, H), lambda i: (0, 0)),
            pl.BlockSpec((H, H), lambda i: (0, 0)),
            pl.BlockSpec((1, H), lambda i: (0, 0)),
        ],
        out_specs=[
            pl.BlockSpec((BM, H), lambda i: (i, 0)),
            pl.BlockSpec((2, H), lambda i: (0, 0)),
        ],
        out_shape=[
            jax.ShapeDtypeStruct((N, H), jnp.float32),
            jax.ShapeDtypeStruct((2, H), jnp.float32),
        ],
    )(m, w1, b1, w2, b2)


def _bn_pool_call(first, m2, cur, st, g, bb, lw, pin, batch_r):
    pin_spec = (pl.BlockSpec((L, OUT), lambda i: (0, 0)) if first
                else pl.BlockSpec((G, OUT), lambda i: (0, 0)))
    return pl.pallas_call(
        functools.partial(_bn_pool_body, first),
        grid=(NBLK,),
        in_specs=[
            pl.BlockSpec((BM, H), lambda i: (i, 0)),
            pl.BlockSpec((BM, H), lambda i: (i, 0)),
            pl.BlockSpec((2, H), lambda i: (0, 0)),
            pl.BlockSpec((1, H), lambda i: (0, 0)),
            pl.BlockSpec((1, H), lambda i: (0, 0)),
            pl.BlockSpec((H, OUT), lambda i: (0, 0)),
            pin_spec,
            pl.BlockSpec((1, 1, BM), lambda i: (i, 0, 0)),
        ],
        out_specs=[
            pl.BlockSpec((BM, H), lambda i: (i, 0)),
            pl.BlockSpec((G, OUT), lambda i: (0, 0)),
        ],
        out_shape=[
            jax.ShapeDtypeStruct((N, H), jnp.float32),
            jax.ShapeDtypeStruct((G, OUT), jnp.float32),
        ],
    )(m2, cur, st, g, bb, lw, pin, batch_r)


def _quarter_layout(cur):
    return cur.reshape(N, NQ, QTR).transpose(1, 0, 2).reshape(NQ * N, QTR)


def _unquarter_layout(m_q):
    return m_q.reshape(NQ, N, QTR).transpose(1, 0, 2).reshape(N, H)


def kernel(x, edge_index, cycle_index, batch, W_emb, b_emb, conv_W1, conv_b1,
           conv_W2, conv_b2, bn_g, bn_b, lin_W, lin_b):
    src = edge_index[0].astype(jnp.int32)
    dst = edge_index[1].astype(jnp.int32)
    pad = EPAD - E
    src_p = jnp.concatenate([src, jnp.zeros((pad,), jnp.int32)])
    dst_p = jnp.concatenate([dst, jnp.full((pad,), N, jnp.int32)])
    src_all = src_p.reshape(NROWS_IDX, CHUNK)
    dst_all = dst_p.reshape(NROWS_IDX, CHUNK)
    batch_r = batch.astype(jnp.int32).reshape(NBLK, 1, BM)

    cur = _emb_call(x, W_emb, b_emb.reshape(1, H))

    pool = lin_b
    for i in range(L):
        cur_q = _quarter_layout(cur)
        m = _unquarter_layout(_sc_agg(cur_q, src_all, dst_all))
        m2, st = _mlp_call(m, conv_W1[i], conv_b1[i].reshape(1, H),
                           conv_W2[i], conv_b2[i].reshape(1, H))
        cur, pool = _bn_pool_call(i == 0, m2, cur, st,
                                  bn_g[i].reshape(1, H), bn_b[i].reshape(1, H),
                                  lin_W[i], pool, batch_r)
    return pool
